# Initial kernel scaffold; baseline (speedup 1.0000x reference)
import jax
import jax.numpy as jnp
from jax.experimental import pallas as pl


def kernel(user_emb, item_emb, edge_index):
    raise NotImplementedError("write your pallas kernel here")


# speedup vs baseline: 2.9266x; 2.9266x over previous
import jax
import jax.numpy as jnp
from jax import lax
from jax.experimental import pallas as pl
from jax.experimental.pallas import tpu as pltpu
from jax.experimental.pallas import tpu_sc as plsc

NU = 10000
NI = 10000
D = 128
E = 320000
NC = 2
NS = 16
L = 16
DH = D // NC
NP = 10112
RPT = NP // NS
G = 128
NG = 160
EPT = G * NG
EPAD = EPT * NS
PAD_NODE = NP - 8
CW = 16
ZR = 64


def _gcn_body(u_hbm, i_hbm, src_hbm, dst_hbm,
              out_u, out_i, h1_u, h1_i,
              src_v, dst_v, gidx, msg, nbuf, zbuf, tbuf, ones,
              acc, cnt_i, cnt_u):
    c = lax.axis_index("c")
    s = lax.axis_index("s")
    row0 = s * RPT
    coff = c * NP

    pltpu.sync_copy(src_hbm.at[pl.ds(s * NG, NG)], src_v)
    pltpu.sync_copy(dst_hbm.at[pl.ds(s * NG, NG)], dst_v)

    def _zrow(r, _):
        for v in range(DH // L):
            zbuf[r, v * L:(v + 1) * L] = jnp.zeros((L,), jnp.float32)
        return 0
    lax.fori_loop(0, ZR, _zrow, 0)

    def _crow(r, _):
        tbuf[r, 0:L] = jnp.zeros((L,), jnp.float32)
        ones[r, 0:L] = jnp.full((L,), 1.0, jnp.float32)
        return 0
    lax.fori_loop(0, G, _crow, 0)

    chunks = []
    off = 0
    while off < RPT:
        chunks.append((off, min(G, RPT - off)))
        off += G

    def _zero_acc():
        off = 0
        while off < RPT:
            n = min(ZR, RPT - off)
            pltpu.sync_copy(zbuf.at[pl.ds(0, n)], acc.at[pl.ds(row0 + off, n)])
            off += n

    def _zero_cnt(cnt):
        for (o, n) in chunks:
            pltpu.sync_copy(tbuf.at[pl.ds(0, n)], cnt.at[pl.ds(row0 + o, n)])

    _zero_acc()
    _zero_cnt(cnt_i)
    _zero_cnt(cnt_u)
    plsc.subcore_barrier()

    def _cnt_step(j, _):
        pltpu.sync_copy(ones, cnt_i.at[dst_v.at[j]], add=True)
        pltpu.sync_copy(ones, cnt_u.at[src_v.at[j]], add=True)
        return 0
    lax.fori_loop(0, NG, _cnt_step, 0)
    plsc.subcore_barrier()

    def _mp_pass(gtab, idx_g, idx_s):
        def _step(j, _):
            for v in range(G // L):
                gidx[v * L:(v + 1) * L] = idx_g[j, v * L:(v + 1) * L] + coff
            pltpu.sync_copy(gtab.at[gidx], msg)
            pltpu.sync_copy(msg, acc.at[idx_s.at[j]], add=True)
            return 0
        lax.fori_loop(0, NG, _step, 0)

    def _normalize(cnt, out, rezero):
        for (o, n) in chunks:
            base = row0 + o
            pltpu.sync_copy(acc.at[pl.ds(base, n)], nbuf.at[pl.ds(0, n)])
            pltpu.sync_copy(cnt.at[pl.ds(base, n)], tbuf.at[pl.ds(0, n)])

            def _r(r, _):
                iv = 1.0 / jnp.maximum(tbuf[r, 0:L], 1.0)
                for v in range(DH // L):
                    nbuf[r, v * L:(v + 1) * L] = nbuf[r, v * L:(v + 1) * L] * iv
                return 0
            lax.fori_loop(0, n, _r, 0)
            pltpu.sync_copy(nbuf.at[pl.ds(0, n)],
                            out.at[pl.ds(coff + base, n)])
        if rezero:
            _zero_acc()

    def _direction(gtab, idx_g, idx_s, cnt, out, rezero):
        _mp_pass(gtab, idx_g, idx_s)
        plsc.subcore_barrier()
        _normalize(cnt, out, rezero)
        plsc.subcore_barrier()

    _direction(u_hbm, src_v, dst_v, cnt_i, h1_i, True)
    _direction(i_hbm, dst_v, src_v, cnt_u, h1_u, True)
    _direction(h1_u, src_v, dst_v, cnt_i, out_i, True)
    _direction(h1_i, dst_v, src_v, cnt_u, out_u, False)


_gcn_call = pl.kernel(
    _gcn_body,
    out_type=(
        jax.ShapeDtypeStruct((NC * NP, DH), jnp.float32),
        jax.ShapeDtypeStruct((NC * NP, DH), jnp.float32),
        jax.ShapeDtypeStruct((NC * NP, DH), jnp.float32),
        jax.ShapeDtypeStruct((NC * NP, DH), jnp.float32),
    ),
    mesh=plsc.VectorSubcoreMesh(core_axis_name="c", subcore_axis_name="s",
                                num_cores=NC, num_subcores=NS),
    scratch_types=(
        pltpu.VMEM((NG, G), jnp.int32),
        pltpu.VMEM((NG, G), jnp.int32),
        pltpu.VMEM((G,), jnp.int32),
        pltpu.VMEM((G, DH), jnp.float32),
        pltpu.VMEM((G, DH), jnp.float32),
        pltpu.VMEM((ZR, DH), jnp.float32),
        pltpu.VMEM((G, CW), jnp.float32),
        pltpu.VMEM((G, CW), jnp.float32),
        pltpu.VMEM_SHARED((NP, DH), jnp.float32),
        pltpu.VMEM_SHARED((NP, CW), jnp.float32),
        pltpu.VMEM_SHARED((NP, CW), jnp.float32),
    ),
    compiler_params=pltpu.CompilerParams(use_tc_tiling_on_sc=False),
)


def kernel(user_emb, item_emb, edge_index):
    src = edge_index[0].astype(jnp.int32)
    dst = edge_index[1].astype(jnp.int32)
    pad = jnp.full((EPAD - E,), PAD_NODE, jnp.int32)
    src2d = jnp.concatenate([src, pad]).reshape(NS * NG, G)
    dst2d = jnp.concatenate([dst, pad]).reshape(NS * NG, G)

    zrows_u = jnp.zeros((NP - NU, D), jnp.float32)
    zrows_i = jnp.zeros((NP - NI, D), jnp.float32)
    up = jnp.concatenate([user_emb, zrows_u], axis=0)
    ip = jnp.concatenate([item_emb, zrows_i], axis=0)
    u_stack = jnp.concatenate([up[:, :DH], up[:, DH:]], axis=0)
    i_stack = jnp.concatenate([ip[:, :DH], ip[:, DH:]], axis=0)

    out_u, out_i, _, _ = _gcn_call(u_stack, i_stack, src2d, dst2d)

    u2 = jnp.concatenate([out_u[:NU], out_u[NP:NP + NU]], axis=1)
    i2 = jnp.concatenate([out_i[:NI], out_i[NP:NP + NI]], axis=1)
    return jnp.concatenate([u2, i2], axis=0)


# speedup vs baseline: 3.7921x; 1.2957x over previous
import jax
import jax.numpy as jnp
from jax import lax
from jax.experimental import pallas as pl
from jax.experimental.pallas import tpu as pltpu
from jax.experimental.pallas import tpu_sc as plsc

NU = 10000
NI = 10000
D = 128
E = 320000
NC = 2
NS = 16
L = 16
DH = D // NC
NP = 10112
RPT = NP // NS
G = 128
NG = 160
EPT = G * NG
EPAD = EPT * NS
PAD_NODE = NP - 8
CW = 16
ZR = 64
NB = 2
CH = 64


def _gcn_body(u_hbm, i_hbm, src_hbm, dst_hbm,
              out_u, out_i, h1_u, h1_i,
              src_v, dst_v, gidx, msg, nbuf, zbuf, tbuf, ones,
              sem0, sem1,
              acc, cnt_i, cnt_u):
    sems = (sem0, sem1)
    c = lax.axis_index("c")
    s = lax.axis_index("s")
    row0 = s * RPT
    coff = c * NP

    pltpu.sync_copy(src_hbm.at[pl.ds(s * NG, NG)], src_v)
    pltpu.sync_copy(dst_hbm.at[pl.ds(s * NG, NG)], dst_v)

    def _zrow(r, _):
        for v in range(DH // L):
            zbuf[r, v * L:(v + 1) * L] = jnp.zeros((L,), jnp.float32)
        return 0
    lax.fori_loop(0, ZR, _zrow, 0)

    def _crow(r, _):
        tbuf[r, 0:L] = jnp.zeros((L,), jnp.float32)
        ones[r, 0:L] = jnp.full((L,), 1.0, jnp.float32)
        return 0
    lax.fori_loop(0, G, _crow, 0)

    chunks = []
    off = 0
    while off < RPT:
        chunks.append((off, min(CH, RPT - off)))
        off += CH

    def _zero_acc():
        off = 0
        while off < RPT:
            n = min(ZR, RPT - off)
            pltpu.sync_copy(zbuf.at[pl.ds(0, n)], acc.at[pl.ds(row0 + off, n)])
            off += n

    def _zero_cnt(cnt):
        for (o, n) in chunks:
            pltpu.sync_copy(tbuf.at[pl.ds(0, n)], cnt.at[pl.ds(row0 + o, n)])

    _zero_acc()
    _zero_cnt(cnt_i)
    _zero_cnt(cnt_u)
    plsc.subcore_barrier()

    def _cnt_step(j, _):
        pltpu.sync_copy(ones, cnt_i.at[dst_v.at[j]], add=True)
        pltpu.sync_copy(ones, cnt_u.at[src_v.at[j]], add=True)
        return 0
    lax.fori_loop(0, NG, _cnt_step, 0)
    plsc.subcore_barrier()

    def _mp_pass(gtab, idx_g, idx_s):
        def _fill(b, g):
            for v in range(G // L):
                gidx[b, v * L:(v + 1) * L] = idx_g[g, v * L:(v + 1) * L] + coff

        def _fire(b):
            pltpu.async_copy(gtab.at[gidx.at[b]], msg.at[b], sems[b])

        def _drain(b):
            pltpu.make_async_copy(gtab.at[gidx.at[b]], msg.at[b],
                                  sems[b]).wait()

        for b in range(NB):
            _fill(b, b)
            _fire(b)

        def _step(i, _):
            for b in range(NB):
                g = i * NB + b
                _drain(b)
                pltpu.sync_copy(msg.at[b], acc.at[idx_s.at[g]], add=True)
                _fill(b, g + NB)
                _fire(b)
            return 0
        lax.fori_loop(0, (NG - NB) // NB, _step, 0)

        for b in range(NB):
            g = NG - NB + b
            _drain(b)
            pltpu.sync_copy(msg.at[b], acc.at[idx_s.at[g]], add=True)

    def _normalize(cnt, out, rezero):
        for (o, n) in chunks:
            base = row0 + o
            pltpu.sync_copy(acc.at[pl.ds(base, n)], nbuf.at[pl.ds(0, n)])
            pltpu.sync_copy(cnt.at[pl.ds(base, n)], tbuf.at[pl.ds(0, n)])

            def _r(r, _):
                iv = 1.0 / jnp.maximum(tbuf[r, 0:L], 1.0)
                for v in range(DH // L):
                    nbuf[r, v * L:(v + 1) * L] = nbuf[r, v * L:(v + 1) * L] * iv
                return 0
            lax.fori_loop(0, n, _r, 0)
            pltpu.sync_copy(nbuf.at[pl.ds(0, n)],
                            out.at[pl.ds(coff + base, n)])
        if rezero:
            _zero_acc()

    def _direction(gtab, idx_g, idx_s, cnt, out, rezero):
        _mp_pass(gtab, idx_g, idx_s)
        plsc.subcore_barrier()
        _normalize(cnt, out, rezero)
        plsc.subcore_barrier()

    _direction(u_hbm, src_v, dst_v, cnt_i, h1_i, True)
    _direction(i_hbm, dst_v, src_v, cnt_u, h1_u, True)
    _direction(h1_u, src_v, dst_v, cnt_i, out_i, True)
    _direction(h1_i, dst_v, src_v, cnt_u, out_u, False)


_gcn_call = pl.kernel(
    _gcn_body,
    out_type=(
        jax.ShapeDtypeStruct((NC * NP, DH), jnp.float32),
        jax.ShapeDtypeStruct((NC * NP, DH), jnp.float32),
        jax.ShapeDtypeStruct((NC * NP, DH), jnp.float32),
        jax.ShapeDtypeStruct((NC * NP, DH), jnp.float32),
    ),
    mesh=plsc.VectorSubcoreMesh(core_axis_name="c", subcore_axis_name="s",
                                num_cores=NC, num_subcores=NS),
    scratch_types=(
        pltpu.VMEM((NG, G), jnp.int32),
        pltpu.VMEM((NG, G), jnp.int32),
        pltpu.VMEM((NB, G), jnp.int32),
        pltpu.VMEM((NB, G, DH), jnp.float32),
        pltpu.VMEM((CH, DH), jnp.float32),
        pltpu.VMEM((ZR, DH), jnp.float32),
        pltpu.VMEM((G, CW), jnp.float32),
        pltpu.VMEM((G, CW), jnp.float32),
        pltpu.SemaphoreType.DMA,
        pltpu.SemaphoreType.DMA,
        pltpu.VMEM_SHARED((NP, DH), jnp.float32),
        pltpu.VMEM_SHARED((NP, CW), jnp.float32),
        pltpu.VMEM_SHARED((NP, CW), jnp.float32),
    ),
    compiler_params=pltpu.CompilerParams(use_tc_tiling_on_sc=False),
)


def kernel(user_emb, item_emb, edge_index):
    src = edge_index[0].astype(jnp.int32)
    dst = edge_index[1].astype(jnp.int32)
    pad = jnp.full((EPAD - E,), PAD_NODE, jnp.int32)
    src2d = jnp.concatenate([src, pad]).reshape(NS * NG, G)
    dst2d = jnp.concatenate([dst, pad]).reshape(NS * NG, G)

    zrows_u = jnp.zeros((NP - NU, D), jnp.float32)
    zrows_i = jnp.zeros((NP - NI, D), jnp.float32)
    up = jnp.concatenate([user_emb, zrows_u], axis=0)
    ip = jnp.concatenate([item_emb, zrows_i], axis=0)
    u_stack = jnp.concatenate([up[:, :DH], up[:, DH:]], axis=0)
    i_stack = jnp.concatenate([ip[:, :DH], ip[:, DH:]], axis=0)

    out_u, out_i, _, _ = _gcn_call(u_stack, i_stack, src2d, dst2d)

    u2 = jnp.concatenate([out_u[:NU], out_u[NP:NP + NU]], axis=1)
    i2 = jnp.concatenate([out_i[:NI], out_i[NP:NP + NI]], axis=1)
    return jnp.concatenate([u2, i2], axis=0)


# speedup vs baseline: 3.8739x; 1.0216x over previous
import jax
import jax.numpy as jnp
from jax import lax
from jax.experimental import pallas as pl
from jax.experimental.pallas import tpu as pltpu
from jax.experimental.pallas import tpu_sc as plsc

NU = 10000
NI = 10000
D = 128
E = 320000
NC = 2
NS = 16
L = 16
DH = D // NC
NP = 10112
RPT = NP // NS
G = 128
NG = 160
EPT = G * NG
EPAD = EPT * NS
PAD_NODE = NP - 8
CW = 16
ZR = 64
NB = 4
IB = 2 * NB
CH = 64


def _gcn_body(u_hbm, i_hbm, idx_hbm,
              out_u, out_i, h1_u, h1_i,
              ibuf, gidx, msg, nbuf, zbuf, tbuf, ones,
              i0, i1, i2, i3, i4, i5, i6, i7,
              g0, g1, g2, g3,
              acc, cnt_i, cnt_u):
    isems = (i0, i1, i2, i3, i4, i5, i6, i7)
    gsems = (g0, g1, g2, g3)
    c = lax.axis_index("c")
    s = lax.axis_index("s")
    row0 = s * RPT
    coff = c * NP
    gbase = s * NG

    def _idx_fire(k, g):
        pltpu.async_copy(idx_hbm.at[gbase + g], ibuf.at[k], isems[k])

    def _idx_wait(k):
        pltpu.make_async_copy(idx_hbm.at[gbase], ibuf.at[k], isems[k]).wait()

    def _zrow(r, _):
        for v in range(DH // L):
            zbuf[r, v * L:(v + 1) * L] = jnp.zeros((L,), jnp.float32)
        return 0
    lax.fori_loop(0, ZR, _zrow, 0)

    def _crow(r, _):
        tbuf[r, 0:L] = jnp.zeros((L,), jnp.float32)
        ones[r, 0:L] = jnp.full((L,), 1.0, jnp.float32)
        return 0
    lax.fori_loop(0, G, _crow, 0)

    chunks = []
    off = 0
    while off < RPT:
        chunks.append((off, min(CH, RPT - off)))
        off += CH

    def _zero_acc():
        off = 0
        while off < RPT:
            n = min(ZR, RPT - off)
            pltpu.sync_copy(zbuf.at[pl.ds(0, n)], acc.at[pl.ds(row0 + off, n)])
            off += n

    def _zero_cnt(cnt):
        for (o, n) in chunks:
            pltpu.sync_copy(tbuf.at[pl.ds(0, n)], cnt.at[pl.ds(row0 + o, n)])

    _zero_acc()
    _zero_cnt(cnt_i)
    _zero_cnt(cnt_u)
    plsc.subcore_barrier()

    for k in range(IB):
        _idx_fire(k, k)

    def _cstep(k, g):
        _idx_wait(k)
        pltpu.sync_copy(ones, cnt_i.at[ibuf.at[k, 1]], add=True)
        pltpu.sync_copy(ones, cnt_u.at[ibuf.at[k, 0]], add=True)

    def _cmain(i, _):
        gb = i * IB
        for k in range(IB):
            _cstep(k, gb + k)
            _idx_fire(k, gb + k + IB)
        return 0
    lax.fori_loop(0, (NG - IB) // IB, _cmain, 0)
    for k in range(IB):
        _cstep(k, NG - IB + k)
    plsc.subcore_barrier()

    def _mp_pass(gtab, gd, sd):
        def _build_fire(k, b):
            for v in range(G // L):
                gidx[b, v * L:(v + 1) * L] = ibuf[k, gd, v * L:(v + 1) * L] \
                    + coff
            pltpu.async_copy(gtab.at[gidx.at[b]], msg.at[b], gsems[b])

        def _gdrain(b):
            pltpu.make_async_copy(gtab.at[gidx.at[b]], msg.at[b],
                                  gsems[b]).wait()

        def _scatter(k, b):
            pltpu.sync_copy(msg.at[b], acc.at[ibuf.at[k, sd]], add=True)

        for k in range(IB):
            _idx_fire(k, k)
        for b in range(NB):
            _idx_wait(b)
            _build_fire(b, b)

        def _main(i, _):
            gb = i * IB
            for k in range(IB):
                b = k % NB
                _gdrain(b)
                _scatter(k, b)
                _idx_fire(k, gb + k + IB)
                k2 = (k + NB) % IB
                _idx_wait(k2)
                _build_fire(k2, b)
            return 0
        lax.fori_loop(0, (NG - IB) // IB, _main, 0)

        for k in range(IB - NB):
            b = k % NB
            _gdrain(b)
            _scatter(k, b)
            k2 = k + NB
            _idx_wait(k2)
            _build_fire(k2, b)
        for k in range(IB - NB, IB):
            b = k % NB
            _gdrain(b)
            _scatter(k, b)

    def _normalize(cnt, out, rezero):
        for (o, n) in chunks:
            base = row0 + o
            pltpu.sync_copy(acc.at[pl.ds(base, n)], nbuf.at[pl.ds(0, n)])
            pltpu.sync_copy(cnt.at[pl.ds(base, n)], tbuf.at[pl.ds(0, n)])

            def _r(r, _):
                iv = 1.0 / jnp.maximum(tbuf[r, 0:L], 1.0)
                for v in range(DH // L):
                    nbuf[r, v * L:(v + 1) * L] = nbuf[r, v * L:(v + 1) * L] * iv
                return 0
            lax.fori_loop(0, n, _r, 0)
            pltpu.sync_copy(nbuf.at[pl.ds(0, n)],
                            out.at[pl.ds(coff + base, n)])
        if rezero:
            _zero_acc()

    def _direction(gtab, gd, sd, cnt, out, rezero):
        _mp_pass(gtab, gd, sd)
        plsc.subcore_barrier()
        _normalize(cnt, out, rezero)
        plsc.subcore_barrier()

    _direction(u_hbm, 0, 1, cnt_i, h1_i, True)
    _direction(i_hbm, 1, 0, cnt_u, h1_u, True)
    _direction(h1_u, 0, 1, cnt_i, out_i, True)
    _direction(h1_i, 1, 0, cnt_u, out_u, False)


_gcn_call = pl.kernel(
    _gcn_body,
    out_type=(
        jax.ShapeDtypeStruct((NC * NP, DH), jnp.float32),
        jax.ShapeDtypeStruct((NC * NP, DH), jnp.float32),
        jax.ShapeDtypeStruct((NC * NP, DH), jnp.float32),
        jax.ShapeDtypeStruct((NC * NP, DH), jnp.float32),
    ),
    mesh=plsc.VectorSubcoreMesh(core_axis_name="c", subcore_axis_name="s",
                                num_cores=NC, num_subcores=NS),
    scratch_types=(
        pltpu.VMEM((IB, 2, G), jnp.int32),
        pltpu.VMEM((NB, G), jnp.int32),
        pltpu.VMEM((NB, G, DH), jnp.float32),
        pltpu.VMEM((CH, DH), jnp.float32),
        pltpu.VMEM((ZR, DH), jnp.float32),
        pltpu.VMEM((G, CW), jnp.float32),
        pltpu.VMEM((G, CW), jnp.float32),
        pltpu.SemaphoreType.DMA,
        pltpu.SemaphoreType.DMA,
        pltpu.SemaphoreType.DMA,
        pltpu.SemaphoreType.DMA,
        pltpu.SemaphoreType.DMA,
        pltpu.SemaphoreType.DMA,
        pltpu.SemaphoreType.DMA,
        pltpu.SemaphoreType.DMA,
        pltpu.SemaphoreType.DMA,
        pltpu.SemaphoreType.DMA,
        pltpu.SemaphoreType.DMA,
        pltpu.SemaphoreType.DMA,
        pltpu.VMEM_SHARED((NP, DH), jnp.float32),
        pltpu.VMEM_SHARED((NP, CW), jnp.float32),
        pltpu.VMEM_SHARED((NP, CW), jnp.float32),
    ),
    compiler_params=pltpu.CompilerParams(use_tc_tiling_on_sc=False),
)


def kernel(user_emb, item_emb, edge_index):
    src = edge_index[0].astype(jnp.int32)
    dst = edge_index[1].astype(jnp.int32)
    pad = jnp.full((EPAD - E,), PAD_NODE, jnp.int32)
    src2d = jnp.concatenate([src, pad]).reshape(NS * NG, G)
    dst2d = jnp.concatenate([dst, pad]).reshape(NS * NG, G)
    idx2 = jnp.stack([src2d, dst2d], axis=1)

    zrows_u = jnp.zeros((NP - NU, D), jnp.float32)
    zrows_i = jnp.zeros((NP - NI, D), jnp.float32)
    up = jnp.concatenate([user_emb, zrows_u], axis=0)
    ip = jnp.concatenate([item_emb, zrows_i], axis=0)
    u_stack = jnp.concatenate([up[:, :DH], up[:, DH:]], axis=0)
    i_stack = jnp.concatenate([ip[:, :DH], ip[:, DH:]], axis=0)

    out_u, out_i, _, _ = _gcn_call(u_stack, i_stack, idx2)

    u2 = jnp.concatenate([out_u[:NU], out_u[NP:NP + NU]], axis=1)
    i2 = jnp.concatenate([out_i[:NI], out_i[NP:NP + NI]], axis=1)
    return jnp.concatenate([u2, i2], axis=0)


# speedup vs baseline: 3.8760x; 1.0005x over previous
import jax
import jax.numpy as jnp
from jax import lax
from jax.experimental import pallas as pl
from jax.experimental.pallas import tpu as pltpu
from jax.experimental.pallas import tpu_sc as plsc

NU = 10000
NI = 10000
D = 128
E = 320000
NC = 2
NS = 16
L = 16
DH = D // NC
NP = 10112
RPT = NP // NS
G = 128
NG = 160
EPT = G * NG
EPAD = EPT * NS
PAD_NODE = NP - 8
CW = 16
ZR = 64
NB = 5
IB = 2 * NB
CH = 64


def _gcn_body(u_hbm, i_hbm, idx_hbm,
              out_u, out_i, h1_u, h1_i,
              ibuf, gidx, msg, nbuf, zbuf, tbuf, ones,
              i0, i1, i2, i3, i4, i5, i6, i7, i8, i9,
              g0, g1, g2, g3, g4,
              acc, cnt_i, cnt_u):
    isems = (i0, i1, i2, i3, i4, i5, i6, i7, i8, i9)
    gsems = (g0, g1, g2, g3, g4)
    c = lax.axis_index("c")
    s = lax.axis_index("s")
    row0 = s * RPT
    coff = c * NP
    gbase = s * NG

    def _idx_fire(k, g):
        pltpu.async_copy(idx_hbm.at[gbase + g], ibuf.at[k], isems[k])

    def _idx_wait(k):
        pltpu.make_async_copy(idx_hbm.at[gbase], ibuf.at[k], isems[k]).wait()

    def _zrow(r, _):
        for v in range(DH // L):
            zbuf[r, v * L:(v + 1) * L] = jnp.zeros((L,), jnp.float32)
        return 0
    lax.fori_loop(0, ZR, _zrow, 0)

    def _crow(r, _):
        tbuf[r, 0:L] = jnp.zeros((L,), jnp.float32)
        ones[r, 0:L] = jnp.full((L,), 1.0, jnp.float32)
        return 0
    lax.fori_loop(0, G, _crow, 0)

    chunks = []
    off = 0
    while off < RPT:
        chunks.append((off, min(CH, RPT - off)))
        off += CH

    def _zero_acc():
        off = 0
        while off < RPT:
            n = min(ZR, RPT - off)
            pltpu.sync_copy(zbuf.at[pl.ds(0, n)], acc.at[pl.ds(row0 + off, n)])
            off += n

    def _zero_cnt(cnt):
        for (o, n) in chunks:
            pltpu.sync_copy(tbuf.at[pl.ds(0, n)], cnt.at[pl.ds(row0 + o, n)])

    _zero_acc()
    _zero_cnt(cnt_i)
    _zero_cnt(cnt_u)
    plsc.subcore_barrier()

    for k in range(IB):
        _idx_fire(k, k)

    def _cstep(k, g):
        _idx_wait(k)
        pltpu.sync_copy(ones, cnt_i.at[ibuf.at[k, 1]], add=True)
        pltpu.sync_copy(ones, cnt_u.at[ibuf.at[k, 0]], add=True)

    def _cmain(i, _):
        gb = i * IB
        for k in range(IB):
            _cstep(k, gb + k)
            _idx_fire(k, gb + k + IB)
        return 0
    lax.fori_loop(0, (NG - IB) // IB, _cmain, 0)
    for k in range(IB):
        _cstep(k, NG - IB + k)
    plsc.subcore_barrier()

    def _mp_pass(gtab, gd, sd):
        def _build_fire(k, b):
            for v in range(G // L):
                gidx[b, v * L:(v + 1) * L] = ibuf[k, gd, v * L:(v + 1) * L] \
                    + coff
            pltpu.async_copy(gtab.at[gidx.at[b]], msg.at[b], gsems[b])

        def _gdrain(b):
            pltpu.make_async_copy(gtab.at[gidx.at[b]], msg.at[b],
                                  gsems[b]).wait()

        def _scatter(k, b):
            pltpu.sync_copy(msg.at[b], acc.at[ibuf.at[k, sd]], add=True)

        for k in range(IB):
            _idx_fire(k, k)
        for b in range(NB):
            _idx_wait(b)
            _build_fire(b, b)

        def _main(i, _):
            gb = i * IB
            for k in range(IB):
                b = k % NB
                _gdrain(b)
                _scatter(k, b)
                _idx_fire(k, gb + k + IB)
                k2 = (k + NB) % IB
                _idx_wait(k2)
                _build_fire(k2, b)
            return 0
        lax.fori_loop(0, (NG - IB) // IB, _main, 0)

        for k in range(IB - NB):
            b = k % NB
            _gdrain(b)
            _scatter(k, b)
            k2 = k + NB
            _idx_wait(k2)
            _build_fire(k2, b)
        for k in range(IB - NB, IB):
            b = k % NB
            _gdrain(b)
            _scatter(k, b)

    def _normalize(cnt, out, rezero):
        for (o, n) in chunks:
            base = row0 + o
            pltpu.sync_copy(acc.at[pl.ds(base, n)], nbuf.at[pl.ds(0, n)])
            pltpu.sync_copy(cnt.at[pl.ds(base, n)], tbuf.at[pl.ds(0, n)])

            def _r(r, _):
                iv = 1.0 / jnp.maximum(tbuf[r, 0:L], 1.0)
                for v in range(DH // L):
                    nbuf[r, v * L:(v + 1) * L] = nbuf[r, v * L:(v + 1) * L] * iv
                return 0
            lax.fori_loop(0, n, _r, 0)
            pltpu.sync_copy(nbuf.at[pl.ds(0, n)],
                            out.at[pl.ds(coff + base, n)])
        if rezero:
            _zero_acc()

    def _direction(gtab, gd, sd, cnt, out, rezero):
        _mp_pass(gtab, gd, sd)
        plsc.subcore_barrier()
        _normalize(cnt, out, rezero)
        plsc.subcore_barrier()

    _direction(u_hbm, 0, 1, cnt_i, h1_i, True)
    _direction(i_hbm, 1, 0, cnt_u, h1_u, True)
    _direction(h1_u, 0, 1, cnt_i, out_i, True)
    _direction(h1_i, 1, 0, cnt_u, out_u, False)


_gcn_call = pl.kernel(
    _gcn_body,
    out_type=(
        jax.ShapeDtypeStruct((NC * NP, DH), jnp.float32),
        jax.ShapeDtypeStruct((NC * NP, DH), jnp.float32),
        jax.ShapeDtypeStruct((NC * NP, DH), jnp.float32),
        jax.ShapeDtypeStruct((NC * NP, DH), jnp.float32),
    ),
    mesh=plsc.VectorSubcoreMesh(core_axis_name="c", subcore_axis_name="s",
                                num_cores=NC, num_subcores=NS),
    scratch_types=(
        pltpu.VMEM((IB, 2, G), jnp.int32),
        pltpu.VMEM((NB, G), jnp.int32),
        pltpu.VMEM((NB, G, DH), jnp.float32),
        pltpu.VMEM((CH, DH), jnp.float32),
        pltpu.VMEM((ZR, DH), jnp.float32),
        pltpu.VMEM((G, CW), jnp.float32),
        pltpu.VMEM((G, CW), jnp.float32),
        pltpu.SemaphoreType.DMA,
        pltpu.SemaphoreType.DMA,
        pltpu.SemaphoreType.DMA,
        pltpu.SemaphoreType.DMA,
        pltpu.SemaphoreType.DMA,
        pltpu.SemaphoreType.DMA,
        pltpu.SemaphoreType.DMA,
        pltpu.SemaphoreType.DMA,
        pltpu.SemaphoreType.DMA,
        pltpu.SemaphoreType.DMA,
        pltpu.SemaphoreType.DMA,
        pltpu.SemaphoreType.DMA,
        pltpu.SemaphoreType.DMA,
        pltpu.SemaphoreType.DMA,
        pltpu.SemaphoreType.DMA,
        pltpu.VMEM_SHARED((NP, DH), jnp.float32),
        pltpu.VMEM_SHARED((NP, CW), jnp.float32),
        pltpu.VMEM_SHARED((NP, CW), jnp.float32),
    ),
    compiler_params=pltpu.CompilerParams(use_tc_tiling_on_sc=False),
)


def kernel(user_emb, item_emb, edge_index):
    src = edge_index[0].astype(jnp.int32)
    dst = edge_index[1].astype(jnp.int32)
    pad = jnp.full((EPAD - E,), PAD_NODE, jnp.int32)
    src2d = jnp.concatenate([src, pad]).reshape(NS * NG, G)
    dst2d = jnp.concatenate([dst, pad]).reshape(NS * NG, G)
    idx2 = jnp.stack([src2d, dst2d], axis=1)

    zrows_u = jnp.zeros((NP - NU, D), jnp.float32)
    zrows_i = jnp.zeros((NP - NI, D), jnp.float32)
    up = jnp.concatenate([user_emb, zrows_u], axis=0)
    ip = jnp.concatenate([item_emb, zrows_i], axis=0)
    u_stack = jnp.concatenate([up[:, :DH], up[:, DH:]], axis=0)
    i_stack = jnp.concatenate([ip[:, :DH], ip[:, DH:]], axis=0)

    out_u, out_i, _, _ = _gcn_call(u_stack, i_stack, idx2)

    u2 = jnp.concatenate([out_u[:NU], out_u[NP:NP + NU]], axis=1)
    i2 = jnp.concatenate([out_i[:NI], out_i[NP:NP + NI]], axis=1)
    return jnp.concatenate([u2, i2], axis=0)


# speedup vs baseline: 3.9884x; 1.0290x over previous
import jax
import jax.numpy as jnp
from jax import lax
from jax.experimental import pallas as pl
from jax.experimental.pallas import tpu as pltpu
from jax.experimental.pallas import tpu_sc as plsc

NU = 10000
NI = 10000
D = 128
E = 320000
NC = 2
NS = 16
L = 16
DH = D // NC
NP = 10112
RPT = NP // NS
G = 128
NG = 160
EPT = G * NG
EPAD = EPT * NS
PAD_NODE = NP - 8
CW = 16
ZR = 64
NB = 5
IB = 2 * NB
CH = 64


def _gcn_body(u_hbm, i_hbm, idx_hbm,
              out_u, out_i, h1_u, h1_i,
              ibuf, gidx, msg, nbuf, zbuf, tbuf, ones,
              i0, i1, i2, i3, i4, i5, i6, i7, i8, i9,
              g0, g1, g2, g3, g4,
              acc, cnt_i, cnt_u):
    isems = (i0, i1, i2, i3, i4, i5, i6, i7, i8, i9)
    gsems = (g0, g1, g2, g3, g4)
    c = lax.axis_index("c")
    s = lax.axis_index("s")
    row0 = s * RPT
    coff = c * NP
    gbase = s * NG

    def _idx_fire(k, g):
        pltpu.async_copy(idx_hbm.at[gbase + g], ibuf.at[k], isems[k])

    def _idx_wait(k):
        pltpu.make_async_copy(idx_hbm.at[gbase], ibuf.at[k], isems[k]).wait()

    def _zrow(r, _):
        for v in range(DH // L):
            zbuf[r, v * L:(v + 1) * L] = jnp.zeros((L,), jnp.float32)
        return 0
    lax.fori_loop(0, ZR, _zrow, 0)

    def _crow(r, _):
        tbuf[r, 0:L] = jnp.zeros((L,), jnp.float32)
        ones[r, 0:L] = jnp.full((L,), 1.0, jnp.float32)
        return 0
    lax.fori_loop(0, G, _crow, 0)

    chunks = []
    off = 0
    while off < RPT:
        chunks.append((off, min(CH, RPT - off)))
        off += CH

    def _zero_acc():
        off = 0
        while off < RPT:
            n = min(ZR, RPT - off)
            pltpu.sync_copy(zbuf.at[pl.ds(0, n)], acc.at[pl.ds(row0 + off, n)])
            off += n

    def _zero_cnt(cnt):
        for (o, n) in chunks:
            pltpu.sync_copy(tbuf.at[pl.ds(0, n)], cnt.at[pl.ds(row0 + o, n)])

    _zero_acc()
    _zero_cnt(cnt_i)
    _zero_cnt(cnt_u)
    plsc.subcore_barrier()

    def _mp_pass(gtab, gd, sd, cnt):
        def _build_fire(k, b):
            for v in range(G // L):
                gidx[b, v * L:(v + 1) * L] = ibuf[k, gd, v * L:(v + 1) * L] \
                    + coff
            pltpu.async_copy(gtab.at[gidx.at[b]], msg.at[b], gsems[b])

        def _gdrain(b):
            pltpu.make_async_copy(gtab.at[gidx.at[b]], msg.at[b],
                                  gsems[b]).wait()

        def _scatter(k, b):
            pltpu.sync_copy(msg.at[b], acc.at[ibuf.at[k, sd]], add=True)
            if cnt is not None:
                pltpu.sync_copy(ones, cnt.at[ibuf.at[k, sd]], add=True)

        for k in range(IB):
            _idx_fire(k, k)
        for b in range(NB):
            _idx_wait(b)
            _build_fire(b, b)

        def _main(i, _):
            gb = i * IB
            for k in range(IB):
                b = k % NB
                _gdrain(b)
                _scatter(k, b)
                _idx_fire(k, gb + k + IB)
                k2 = (k + NB) % IB
                _idx_wait(k2)
                _build_fire(k2, b)
            return 0
        lax.fori_loop(0, (NG - IB) // IB, _main, 0)

        for k in range(IB - NB):
            b = k % NB
            _gdrain(b)
            _scatter(k, b)
            k2 = k + NB
            _idx_wait(k2)
            _build_fire(k2, b)
        for k in range(IB - NB, IB):
            b = k % NB
            _gdrain(b)
            _scatter(k, b)

    def _normalize(cnt, out, rezero):
        for (o, n) in chunks:
            base = row0 + o
            pltpu.sync_copy(acc.at[pl.ds(base, n)], nbuf.at[pl.ds(0, n)])
            pltpu.sync_copy(cnt.at[pl.ds(base, n)], tbuf.at[pl.ds(0, n)])

            def _r(r, _):
                iv = 1.0 / jnp.maximum(tbuf[r, 0:L], 1.0)
                for v in range(DH // L):
                    nbuf[r, v * L:(v + 1) * L] = nbuf[r, v * L:(v + 1) * L] * iv
                return 0
            lax.fori_loop(0, n, _r, 0)
            pltpu.sync_copy(nbuf.at[pl.ds(0, n)],
                            out.at[pl.ds(coff + base, n)])
        if rezero:
            _zero_acc()

    def _direction(gtab, gd, sd, cnt, out, rezero, count):
        _mp_pass(gtab, gd, sd, cnt if count else None)
        plsc.subcore_barrier()
        _normalize(cnt, out, rezero)
        plsc.subcore_barrier()

    _direction(u_hbm, 0, 1, cnt_i, h1_i, True, True)
    _direction(i_hbm, 1, 0, cnt_u, h1_u, True, True)
    _direction(h1_u, 0, 1, cnt_i, out_i, True, False)
    _direction(h1_i, 1, 0, cnt_u, out_u, False, False)


_gcn_call = pl.kernel(
    _gcn_body,
    out_type=(
        jax.ShapeDtypeStruct((NC * NP, DH), jnp.float32),
        jax.ShapeDtypeStruct((NC * NP, DH), jnp.float32),
        jax.ShapeDtypeStruct((NC * NP, DH), jnp.float32),
        jax.ShapeDtypeStruct((NC * NP, DH), jnp.float32),
    ),
    mesh=plsc.VectorSubcoreMesh(core_axis_name="c", subcore_axis_name="s",
                                num_cores=NC, num_subcores=NS),
    scratch_types=(
        pltpu.VMEM((IB, 2, G), jnp.int32),
        pltpu.VMEM((NB, G), jnp.int32),
        pltpu.VMEM((NB, G, DH), jnp.float32),
        pltpu.VMEM((CH, DH), jnp.float32),
        pltpu.VMEM((ZR, DH), jnp.float32),
        pltpu.VMEM((G, CW), jnp.float32),
        pltpu.VMEM((G, CW), jnp.float32),
        pltpu.SemaphoreType.DMA,
        pltpu.SemaphoreType.DMA,
        pltpu.SemaphoreType.DMA,
        pltpu.SemaphoreType.DMA,
        pltpu.SemaphoreType.DMA,
        pltpu.SemaphoreType.DMA,
        pltpu.SemaphoreType.DMA,
        pltpu.SemaphoreType.DMA,
        pltpu.SemaphoreType.DMA,
        pltpu.SemaphoreType.DMA,
        pltpu.SemaphoreType.DMA,
        pltpu.SemaphoreType.DMA,
        pltpu.SemaphoreType.DMA,
        pltpu.SemaphoreType.DMA,
        pltpu.SemaphoreType.DMA,
        pltpu.VMEM_SHARED((NP, DH), jnp.float32),
        pltpu.VMEM_SHARED((NP, CW), jnp.float32),
        pltpu.VMEM_SHARED((NP, CW), jnp.float32),
    ),
    compiler_params=pltpu.CompilerParams(use_tc_tiling_on_sc=False),
)


def kernel(user_emb, item_emb, edge_index):
    src = edge_index[0].astype(jnp.int32)
    dst = edge_index[1].astype(jnp.int32)
    pad = jnp.full((EPAD - E,), PAD_NODE, jnp.int32)
    src2d = jnp.concatenate([src, pad]).reshape(NS * NG, G)
    dst2d = jnp.concatenate([dst, pad]).reshape(NS * NG, G)
    idx2 = jnp.stack([src2d, dst2d], axis=1)

    zrows_u = jnp.zeros((NP - NU, D), jnp.float32)
    zrows_i = jnp.zeros((NP - NI, D), jnp.float32)
    up = jnp.concatenate([user_emb, zrows_u], axis=0)
    ip = jnp.concatenate([item_emb, zrows_i], axis=0)
    u_stack = jnp.concatenate([up[:, :DH], up[:, DH:]], axis=0)
    i_stack = jnp.concatenate([ip[:, :DH], ip[:, DH:]], axis=0)

    out_u, out_i, _, _ = _gcn_call(u_stack, i_stack, idx2)

    u2 = jnp.concatenate([out_u[:NU], out_u[NP:NP + NU]], axis=1)
    i2 = jnp.concatenate([out_i[:NI], out_i[NP:NP + NI]], axis=1)
    return jnp.concatenate([u2, i2], axis=0)
